# Initial kernel scaffold; baseline (speedup 1.0000x reference)
import jax
import jax.numpy as jnp
from jax.experimental import pallas as pl


def kernel(density_nu_l0, density_nu_l1, density_nu_l2, density_nu_l3, density_1_l0, density_1_l1, density_1_l2, density_1_l3):
    raise NotImplementedError("write your pallas kernel here")


# speedup vs baseline: 1.7307x; 1.7307x over previous
import functools

import jax
import jax.numpy as jnp
import numpy as np
from jax.experimental import pallas as pl

L_MAX = 3
Q = 32
PAIR = Q * Q


def _ps_kernel(a0, a1, a2, a3, b0, b1, b2, b3, out_ref):
    j_div = jax.lax.broadcasted_iota(jnp.int32, (Q, PAIR), 1) // Q
    j_mod = jax.lax.broadcasted_iota(jnp.int32, (Q, PAIR), 1) % Q
    row = jax.lax.broadcasted_iota(jnp.int32, (Q, PAIR), 0)
    R = (j_div == row).astype(jnp.float32)
    T = (j_mod == row).astype(jnp.float32)

    a_refs = (a0, a1, a2, a3)
    b_refs = (b0, b1, b2, b3)
    for l in range(L_MAX + 1):
        cg = np.float32(1.0 / np.sqrt(2 * l + 1))
        acc = None
        for m in range(2 * l + 1):
            am = a_refs[l][:, m, :]
            bm = b_refs[l][:, m, :]
            ar = jnp.dot(am, R, preferred_element_type=jnp.float32)
            br = jnp.dot(bm, T, preferred_element_type=jnp.float32)
            term = ar * br
            acc = term if acc is None else acc + term
        out_ref[:, l * PAIR:(l + 1) * PAIR] = acc * cg


@functools.partial(jax.jit, static_argnames=())
def kernel(density_nu_l0, density_nu_l1, density_nu_l2, density_nu_l3,
           density_1_l0, density_1_l1, density_1_l2, density_1_l3):
    n = density_nu_l0.shape[0]
    sb = 256
    grid = (n // sb,)

    in_specs = []
    for l in range(L_MAX + 1):
        spec = pl.BlockSpec((sb, 2 * l + 1, Q), lambda i: (i, 0, 0))
        in_specs.append(spec)
    in_specs = in_specs + in_specs

    out = pl.pallas_call(
        _ps_kernel,
        grid=grid,
        in_specs=in_specs,
        out_specs=pl.BlockSpec((sb, (L_MAX + 1) * PAIR), lambda i: (i, 0)),
        out_shape=jax.ShapeDtypeStruct((n, (L_MAX + 1) * PAIR), jnp.float32),
    )(density_nu_l0, density_nu_l1, density_nu_l2, density_nu_l3,
      density_1_l0, density_1_l1, density_1_l2, density_1_l3)
    return out


# speedup vs baseline: 2.5256x; 1.4593x over previous
import functools

import jax
import jax.numpy as jnp
import numpy as np
from jax.experimental import pallas as pl

L_MAX = 3
Q = 32
PAIR = Q * Q


def _expansion_consts():
    rbigs, tbigs = [], []
    for l in range(L_MAX + 1):
        ml = 2 * l + 1
        rb = np.zeros((ml * Q, ml * PAIR), dtype=np.float32)
        tb = np.zeros((ml * Q, ml * PAIR), dtype=np.float32)
        for m in range(ml):
            for q in range(Q):
                rb[m * Q + q, m * PAIR + q * Q:m * PAIR + (q + 1) * Q] = 1.0
                tb[m * Q + q, m * PAIR + q:m * PAIR + PAIR:Q] = 1.0
        rbigs.append(rb.astype(jnp.bfloat16))
        tbigs.append(tb.astype(jnp.bfloat16))
    return rbigs, tbigs


_RBIGS, _TBIGS = _expansion_consts()


def _ps_kernel(a0, a1, a2, a3, b0, b1, b2, b3,
               r0, r1, r2, r3, t0, t1, t2, t3, out_ref):
    a_refs = (a0, a1, a2, a3)
    b_refs = (b0, b1, b2, b3)
    r_refs = (r0, r1, r2, r3)
    t_refs = (t0, t1, t2, t3)
    for l in range(L_MAX + 1):
        ml = 2 * l + 1
        cg = np.float32(1.0 / np.sqrt(ml))
        a = a_refs[l][...].astype(jnp.bfloat16)
        b = b_refs[l][...].astype(jnp.bfloat16)
        ar = jnp.dot(a, r_refs[l][...], preferred_element_type=jnp.float32)
        bt = jnp.dot(b, t_refs[l][...], preferred_element_type=jnp.float32)
        prod = ar * bt
        acc = prod[:, :PAIR]
        for m in range(1, ml):
            acc = acc + prod[:, m * PAIR:(m + 1) * PAIR]
        out_ref[:, l * PAIR:(l + 1) * PAIR] = acc * cg


@functools.partial(jax.jit, static_argnames=())
def kernel(density_nu_l0, density_nu_l1, density_nu_l2, density_nu_l3,
           density_1_l0, density_1_l1, density_1_l2, density_1_l3):
    n = density_nu_l0.shape[0]
    sb = 256
    grid = (n // sb,)

    nus = (density_nu_l0, density_nu_l1, density_nu_l2, density_nu_l3)
    d1s = (density_1_l0, density_1_l1, density_1_l2, density_1_l3)
    nus = tuple(x.reshape(n, -1) for x in nus)
    d1s = tuple(x.reshape(n, -1) for x in d1s)

    in_specs = []
    for l in range(L_MAX + 1):
        in_specs.append(pl.BlockSpec((sb, (2 * l + 1) * Q), lambda i: (i, 0)))
    in_specs = in_specs + in_specs
    for mats in (_RBIGS, _TBIGS):
        for mat in mats:
            in_specs.append(pl.BlockSpec(mat.shape, lambda i: (0, 0)))

    out = pl.pallas_call(
        _ps_kernel,
        grid=grid,
        in_specs=in_specs,
        out_specs=pl.BlockSpec((sb, (L_MAX + 1) * PAIR), lambda i: (i, 0)),
        out_shape=jax.ShapeDtypeStruct((n, (L_MAX + 1) * PAIR), jnp.float32),
    )(*nus, *d1s, *_RBIGS, *_TBIGS)
    return out
